# baseline (speedup 1.0000x reference)
import functools

import jax
import jax.numpy as jnp
from jax import lax
from jax.experimental import pallas as pl
from jax.experimental.pallas import tpu as pltpu
from jax.experimental.pallas import tpu_sc as plsc

B = 16384
D = 64
N_ROWS = 100000
NC = 2
NS = 16
NW = NC * NS
BPW = B // NW
CHUNK = 128
NCHUNK = BPW // CHUNK


def _sc_gather(idx1_2d, idx2_2d, table1p, table2p):
    mesh = plsc.VectorSubcoreMesh(core_axis_name="c", subcore_axis_name="s")

    @functools.partial(
        pl.kernel,
        out_type=(
            jax.ShapeDtypeStruct((B, 2 * D), jnp.float32),
            jax.ShapeDtypeStruct((B, 2 * D), jnp.float32),
        ),
        mesh=mesh,
        scratch_types=[
            pltpu.VMEM((NCHUNK, CHUNK), jnp.int32),
            pltpu.VMEM((NCHUNK, CHUNK), jnp.int32),
            pltpu.VMEM((BPW, 2 * D), jnp.float32),
            pltpu.SemaphoreType.DMA,
        ],
        compiler_params=pltpu.CompilerParams(use_tc_tiling_on_sc=True),
    )
    def k(t1_hbm, t2_hbm, i1_hbm, i2_hbm, o1_hbm, o2_hbm,
          i1_v, i2_v, r_v, sem):
        wid = lax.axis_index("s") * NC + lax.axis_index("c")
        base = wid * BPW
        crow = wid * NCHUNK
        pltpu.sync_copy(i1_hbm.at[pl.ds(crow, NCHUNK)], i1_v)
        pltpu.sync_copy(i2_hbm.at[pl.ds(crow, NCHUNK)], i2_v)
        copies = []
        for j in range(NCHUNK):
            copies.append(pltpu.async_copy(
                t1_hbm.at[i1_v.at[j]], r_v.at[pl.ds(j * CHUNK, CHUNK)], sem))
        for c in copies:
            c.wait()
        pltpu.sync_copy(r_v, o1_hbm.at[pl.ds(base, BPW)])
        copies = []
        for j in range(NCHUNK):
            copies.append(pltpu.async_copy(
                t2_hbm.at[i2_v.at[j]], r_v.at[pl.ds(j * CHUNK, CHUNK)], sem))
        for c in copies:
            c.wait()
        pltpu.sync_copy(r_v, o2_hbm.at[pl.ds(base, BPW)])

    return k(table1p, table2p, idx1_2d, idx2_2d)


def _select_half(xp, par):
    m = (par[:, None] > 0).astype(jnp.float32)
    return xp[:, :D] * (1.0 - m) + xp[:, D:] * m


def _tc_mlp_body(x1_ref, x2_ref, p1_ref, p2_ref, w1_ref, b1_ref, w2_ref,
                 b2_ref, out_ref):
    x1 = _select_half(x1_ref[...], p1_ref[...])
    x2 = _select_half(x2_ref[...], p2_ref[...])
    n1 = jnp.sqrt(jnp.sum(x1 * x1, axis=1, keepdims=True))
    n2 = jnp.sqrt(jnp.sum(x2 * x2, axis=1, keepdims=True))
    xn = jnp.concatenate(
        [x1 / jnp.maximum(n1, 1e-12), x2 / jnp.maximum(n2, 1e-12)], axis=1)
    h = lax.dot_general(xn, w1_ref[...], (((1,), (1,)), ((), ())),
                        preferred_element_type=jnp.float32)
    h = jnp.maximum(h + b1_ref[...][None, :], 0.0)
    out = jnp.sum(h * w2_ref[...], axis=1, keepdims=True)
    out_ref[...] = jax.nn.sigmoid(out + b2_ref[0])


def _tc_mlp(x1, x2, par1, par2, W1, b1, W2, b2):
    blk = 2048
    grid = (B // blk,)
    return pl.pallas_call(
        _tc_mlp_body,
        grid=grid,
        in_specs=[
            pl.BlockSpec((blk, 2 * D), lambda i: (i, 0)),
            pl.BlockSpec((blk, 2 * D), lambda i: (i, 0)),
            pl.BlockSpec((blk,), lambda i: (i,)),
            pl.BlockSpec((blk,), lambda i: (i,)),
            pl.BlockSpec((D, 2 * D), lambda i: (0, 0)),
            pl.BlockSpec((D,), lambda i: (0,)),
            pl.BlockSpec((1, D), lambda i: (0, 0)),
            pl.BlockSpec(memory_space=pltpu.SMEM),
        ],
        out_specs=pl.BlockSpec((blk, 1), lambda i: (i, 0)),
        out_shape=jax.ShapeDtypeStruct((B, 1), jnp.float32),
    )(x1, x2, par1, par2, W1, b1, W2, b2)


def kernel(user_ids, item_ids, session_idx, action_type_item_idx,
           user_table, item_table, item_table2, W1, b1, W2, b2):
    del user_ids, session_idx, user_table
    idx1 = item_ids.astype(jnp.int32)
    idx2 = action_type_item_idx.astype(jnp.int32)
    i1p = (idx1 >> 1).reshape(B // CHUNK, CHUNK)
    i2p = (idx2 >> 1).reshape(B // CHUNK, CHUNK)
    par1 = idx1 & 1
    par2 = idx2 & 1
    t1p = item_table.reshape(N_ROWS // 2, 2 * D)
    t2p = item_table2.reshape(N_ROWS // 2, 2 * D)
    x1, x2 = _sc_gather(i1p, i2p, t1p, t2p)
    return _tc_mlp(x1, x2, par1, par2, W1, b1, W2, b2)


# speedup vs baseline: 1.3894x; 1.3894x over previous
import functools

import jax
import jax.numpy as jnp
from jax import lax
from jax.experimental import pallas as pl
from jax.experimental.pallas import tpu as pltpu
from jax.experimental.pallas import tpu_sc as plsc

B = 16384
D = 64
N_ROWS = 100000
NC = 2
NS = 16
NW = NC * NS
BPW = B // NW
CHUNK = 128
NCHUNK = BPW // CHUNK


def _sc_gather(idx1_2d, idx2_2d, table1p, table2p):
    mesh = plsc.VectorSubcoreMesh(core_axis_name="c", subcore_axis_name="s")

    @functools.partial(
        pl.kernel,
        out_type=(
            jax.ShapeDtypeStruct((B, 2 * D), jnp.float32),
            jax.ShapeDtypeStruct((B, 2 * D), jnp.float32),
        ),
        mesh=mesh,
        scratch_types=[
            pltpu.VMEM((NCHUNK, CHUNK), jnp.int32),
            pltpu.VMEM((NCHUNK, CHUNK), jnp.int32),
            pltpu.VMEM((BPW, 2 * D), jnp.float32),
            pltpu.SemaphoreType.DMA,
        ],
        compiler_params=pltpu.CompilerParams(use_tc_tiling_on_sc=True),
    )
    def k(t1_hbm, t2_hbm, i1_hbm, i2_hbm, o1_hbm, o2_hbm,
          i1_v, i2_v, r_v, sem):
        wid = lax.axis_index("s") * NC + lax.axis_index("c")
        base = wid * BPW
        crow = wid * NCHUNK
        pltpu.sync_copy(i1_hbm.at[pl.ds(crow, NCHUNK)], i1_v)
        pltpu.sync_copy(i2_hbm.at[pl.ds(crow, NCHUNK)], i2_v)
        copies = []
        for j in range(NCHUNK):
            copies.append(pltpu.async_copy(
                t1_hbm.at[i1_v.at[j]], r_v.at[pl.ds(j * CHUNK, CHUNK)], sem))
        for c in copies:
            c.wait()
        pltpu.sync_copy(r_v, o1_hbm.at[pl.ds(base, BPW)])
        copies = []
        for j in range(NCHUNK):
            copies.append(pltpu.async_copy(
                t2_hbm.at[i2_v.at[j]], r_v.at[pl.ds(j * CHUNK, CHUNK)], sem))
        for c in copies:
            c.wait()
        pltpu.sync_copy(r_v, o2_hbm.at[pl.ds(base, BPW)])

    return k(table1p, table2p, idx1_2d, idx2_2d)


TBLK = 1024
TGRID = 49
SPLIT = TGRID * TBLK
NPACK = SPLIT


def _tc_detile_body(x1a_ref, x1b_ref, x2a_ref, x2b_ref, o1_ref, o2_ref):
    o1_ref[:, :D] = x1a_ref[...].T
    o1_ref[:, D:] = x1b_ref[...].T
    o2_ref[:, :D] = x2a_ref[...].T
    o2_ref[:, D:] = x2b_ref[...].T


def _tc_detile_pair(t1t, t2t):
    return pl.pallas_call(
        _tc_detile_body,
        grid=(TGRID,),
        in_specs=[
            pl.BlockSpec((D, TBLK), lambda i: (0, i)),
            pl.BlockSpec((D, TBLK), lambda i: (0, i + TGRID)),
            pl.BlockSpec((D, TBLK), lambda i: (0, i)),
            pl.BlockSpec((D, TBLK), lambda i: (0, i + TGRID)),
        ],
        out_specs=[
            pl.BlockSpec((TBLK, 2 * D), lambda i: (i, 0)),
            pl.BlockSpec((TBLK, 2 * D), lambda i: (i, 0)),
        ],
        out_shape=(
            jax.ShapeDtypeStruct((NPACK, 2 * D), jnp.float32),
            jax.ShapeDtypeStruct((NPACK, 2 * D), jnp.float32),
        ),
    )(t1t, t1t, t2t, t2t)


def _select_half(xp, par):
    m = (par[:, None] > 0).astype(jnp.float32)
    return xp[:, :D] * (1.0 - m) + xp[:, D:] * m


def _tc_mlp_body(x1_ref, x2_ref, p1_ref, p2_ref, w1_ref, b1_ref, w2_ref,
                 b2_ref, out_ref):
    x1 = _select_half(x1_ref[...], p1_ref[...])
    x2 = _select_half(x2_ref[...], p2_ref[...])
    n1 = jnp.sqrt(jnp.sum(x1 * x1, axis=1, keepdims=True))
    n2 = jnp.sqrt(jnp.sum(x2 * x2, axis=1, keepdims=True))
    xn = jnp.concatenate(
        [x1 / jnp.maximum(n1, 1e-12), x2 / jnp.maximum(n2, 1e-12)], axis=1)
    h = lax.dot_general(xn, w1_ref[...], (((1,), (1,)), ((), ())),
                        preferred_element_type=jnp.float32)
    h = jnp.maximum(h + b1_ref[...][None, :], 0.0)
    out = jnp.sum(h * w2_ref[...], axis=1, keepdims=True)
    out_ref[...] = jax.nn.sigmoid(out + b2_ref[0])


def _tc_mlp(x1, x2, par1, par2, W1, b1, W2, b2):
    blk = 2048
    grid = (B // blk,)
    return pl.pallas_call(
        _tc_mlp_body,
        grid=grid,
        in_specs=[
            pl.BlockSpec((blk, 2 * D), lambda i: (i, 0)),
            pl.BlockSpec((blk, 2 * D), lambda i: (i, 0)),
            pl.BlockSpec((blk,), lambda i: (i,)),
            pl.BlockSpec((blk,), lambda i: (i,)),
            pl.BlockSpec((D, 2 * D), lambda i: (0, 0)),
            pl.BlockSpec((D,), lambda i: (0,)),
            pl.BlockSpec((1, D), lambda i: (0, 0)),
            pl.BlockSpec(memory_space=pltpu.SMEM),
        ],
        out_specs=pl.BlockSpec((blk, 1), lambda i: (i, 0)),
        out_shape=jax.ShapeDtypeStruct((B, 1), jnp.float32),
    )(x1, x2, par1, par2, W1, b1, W2, b2)


def kernel(user_ids, item_ids, session_idx, action_type_item_idx,
           user_table, item_table, item_table2, W1, b1, W2, b2):
    del user_ids, session_idx, user_table
    idx1 = item_ids.astype(jnp.int32)
    idx2 = action_type_item_idx.astype(jnp.int32)
    par1 = (idx1 >= SPLIT).astype(jnp.int32)
    par2 = (idx2 >= SPLIT).astype(jnp.int32)
    i1p = (idx1 - par1 * SPLIT).reshape(B // CHUNK, CHUNK)
    i2p = (idx2 - par2 * SPLIT).reshape(B // CHUNK, CHUNK)
    t1p, t2p = _tc_detile_pair(item_table.T, item_table2.T)
    x1, x2 = _sc_gather(i1p, i2p, t1p, t2p)
    return _tc_mlp(x1, x2, par1, par2, W1, b1, W2, b2)
